# baseline (speedup 1.0000x reference)
import functools

import jax
import jax.numpy as jnp
from jax import lax
from jax.experimental import pallas as pl
from jax.experimental.pallas import tpu as pltpu
from jax.experimental.pallas import tpu_sc as plsc

EMB = 32
BATCH = 16384
NUM_AGE = 7
NUM_GENDERS = 2
NUM_OCC = 21
NUM_GENRES = 18
HID = 128
PADW = 128
PACK = PADW // EMB

_NC = 2
_NS = 16
_NW = _NC * _NS
_B_PER_W = BATCH // _NW
_CHUNK = 128
_NCHUNK = _B_PER_W // _CHUNK


def _gather_body(table_hbm, idx_hbm, out_hbm, idx_v, rows_v, sem):
    wid = lax.axis_index("s") * _NC + lax.axis_index("c")
    base = wid * _B_PER_W
    pltpu.sync_copy(idx_hbm.at[pl.ds(base, _B_PER_W)], idx_v)
    copies = [
        pltpu.async_copy(
            table_hbm.at[idx_v.at[pl.ds(j * _CHUNK, _CHUNK)]],
            rows_v.at[pl.ds(j * _CHUNK, _CHUNK)],
            sem,
        )
        for j in range(_NCHUNK)
    ]
    for c in copies:
        c.wait()
    pltpu.sync_copy(rows_v, out_hbm.at[pl.ds(base, _B_PER_W)])


def _movie_gather(table128, row_idx):
    mesh = plsc.VectorSubcoreMesh(core_axis_name="c", subcore_axis_name="s")
    k = pl.kernel(
        _gather_body,
        mesh=mesh,
        out_type=jax.ShapeDtypeStruct((BATCH, PADW), jnp.float32),
        scratch_types=[
            pltpu.VMEM((_B_PER_W,), jnp.int32),
            pltpu.VMEM((_B_PER_W, PADW), jnp.float32),
            pltpu.SemaphoreType.DMA,
        ],
    )
    return k(table128, row_idx)


_B_BLK = 2048
_GRID = BATCH // _B_BLK


def _mlp_body(mov_ref, gen_ref, q_ref, age_ref, gdr_ref, occ_ref,
              aget_ref, gdrt_ref, occt_ref, gw_ref, gb_ref,
              w1_ref, b1_ref, w2_ref, b2_ref, out_ref):
    f32 = jnp.float32
    W1 = w1_ref[...]
    Wa = W1[0:32]
    Wm = W1[32:64]
    Wg = W1[64:96]
    Wo = W1[96:128]
    Wgen = W1[128:160]

    def onehot(idx_col, n):
        iota = lax.broadcasted_iota(jnp.int32, (_B_BLK, n), 1)
        return (idx_col == iota).astype(f32)

    dot = functools.partial(jnp.dot, preferred_element_type=f32)

    q = q_ref[...]
    mov4 = mov_ref[...]
    mov = jnp.where(q == 0, mov4[:, 0:32], 0.0)
    for k in range(1, PACK):
        mov += jnp.where(q == k, mov4[:, 32 * k:32 * (k + 1)], 0.0)

    acc = dot(mov, Wm)
    acc += dot(onehot(age_ref[...], NUM_AGE), dot(aget_ref[...], Wa))
    acc += dot(onehot(gdr_ref[...], NUM_GENDERS), dot(gdrt_ref[...], Wg))
    acc += dot(onehot(occ_ref[...], NUM_OCC), dot(occt_ref[...], Wo))
    acc += dot(gen_ref[...], dot(gw_ref[...], Wgen))
    acc += dot(gb_ref[...], Wgen) + b1_ref[...]
    h = jnp.maximum(acc, 0.0)
    out_ref[...] = dot(h, w2_ref[...]) + b2_ref[...]


def _mlp(movie4, genres, q2, age2, gdr2, occ2,
         age_table, gender_table, occupation_table,
         genre_W, genre_b, fc1_W, fc1_b, fc2_W, fc2_b,
         interpret=False):
    batch_spec = lambda d: pl.BlockSpec((_B_BLK, d), lambda i: (i, 0))
    full_spec = lambda a, b: pl.BlockSpec((a, b), lambda i: (0, 0))
    out2 = pl.pallas_call(
        _mlp_body,
        grid=(_GRID,),
        in_specs=[
            batch_spec(PADW),
            batch_spec(NUM_GENRES),
            batch_spec(1),
            batch_spec(1),
            batch_spec(1),
            batch_spec(1),
            full_spec(NUM_AGE, EMB),
            full_spec(NUM_GENDERS, EMB),
            full_spec(NUM_OCC, EMB),
            full_spec(NUM_GENRES, EMB),
            full_spec(1, EMB),
            full_spec(5 * EMB, HID),
            full_spec(1, HID),
            full_spec(HID, 1),
            full_spec(1, 1),
        ],
        out_specs=pl.BlockSpec((_B_BLK, 1), lambda i: (i, 0)),
        out_shape=jax.ShapeDtypeStruct((BATCH, 1), jnp.float32),
        interpret=interpret,
    )(movie4, genres, q2, age2, gdr2, occ2,
      age_table, gender_table, occupation_table,
      genre_W, genre_b.reshape(1, EMB),
      fc1_W, fc1_b.reshape(1, HID), fc2_W, fc2_b.reshape(1, 1))
    return out2[:, 0]


def kernel(age_group, movie, gender, occupation, genres,
           movie_table, gender_table, occupation_table, age_table,
           genre_W, genre_b, fc1_W, fc1_b, fc2_W, fc2_b):
    i32 = jnp.int32
    movie = movie.astype(i32)
    table128 = movie_table.reshape(movie_table.shape[0] // PACK, PADW)
    movie4 = _movie_gather(table128, movie // PACK)
    return _mlp(
        movie4,
        genres.astype(jnp.float32),
        (movie % PACK).reshape(BATCH, 1),
        age_group.astype(i32).reshape(BATCH, 1),
        gender.astype(i32).reshape(BATCH, 1),
        occupation.astype(i32).reshape(BATCH, 1),
        age_table, gender_table, occupation_table,
        genre_W, genre_b, fc1_W, fc1_b, fc2_W, fc2_b,
    )


# speedup vs baseline: 1.7699x; 1.7699x over previous
import functools

import jax
import jax.numpy as jnp
from jax import lax
from jax.experimental import pallas as pl
from jax.experimental.pallas import tpu as pltpu
from jax.experimental.pallas import tpu_sc as plsc

EMB = 32
BATCH = 16384
NUM_AGE = 7
NUM_GENDERS = 2
NUM_OCC = 21
NUM_GENRES = 18
HID = 128

_NC = 2
_NS = 16
_NW = _NC * _NS
_B_PER_W = BATCH // _NW


def _gather_body(tbl_hbm, idx_hbm, out_hbm, idx_v, rows_v, sem):
    wid = lax.axis_index("s") * _NC + lax.axis_index("c")
    base = wid * _B_PER_W
    pltpu.sync_copy(idx_hbm.at[pl.ds(base, _B_PER_W)], idx_v)

    @pl.loop(0, _B_PER_W // 16)
    def _fire(g):
        v = idx_v[pl.ds(g * 16, 16)]
        for k in range(16):
            pltpu.make_async_copy(
                tbl_hbm.at[pl.ds(v[k], 1)],
                rows_v.at[pl.ds(g * 16 + k, 1)],
                sem,
            ).start()

    pltpu.make_async_copy(
        tbl_hbm.at[pl.ds(0, _B_PER_W)], rows_v, sem
    ).wait()
    pltpu.sync_copy(rows_v, out_hbm.at[pl.ds(base, _B_PER_W)])


def _movie_gather(table, idx):
    mesh = plsc.VectorSubcoreMesh(core_axis_name="c", subcore_axis_name="s")
    k = pl.kernel(
        _gather_body,
        mesh=mesh,
        out_type=jax.ShapeDtypeStruct((BATCH, EMB), jnp.float32),
        scratch_types=[
            pltpu.VMEM((_B_PER_W,), jnp.int32),
            pltpu.VMEM((_B_PER_W, EMB), jnp.float32),
            pltpu.SemaphoreType.DMA,
        ],
    )
    return k(table, idx)


_B_BLK = 2048
_GRID = BATCH // _B_BLK


def _mlp_body(mov_ref, genT_ref, age_ref, gdr_ref, occ_ref,
              aget_ref, gdrt_ref, occt_ref, gw_ref, gb_ref,
              w1_ref, b1_ref, w2_ref, b2_ref, out_ref):
    f32 = jnp.float32
    W1 = w1_ref[...]
    Wa = W1[0:32]
    Wm = W1[32:64]
    Wg = W1[64:96]
    Wo = W1[96:128]
    Wgen = W1[128:160]

    def onehot_t(idx_row, n):
        iota = lax.broadcasted_iota(jnp.int32, (n, _B_BLK), 0)
        return (idx_row == iota).astype(f32)

    dot = functools.partial(jnp.dot, preferred_element_type=f32)

    def tdot(lhs_t, rhs):
        return lax.dot_general(
            lhs_t, rhs, (((0,), (0,)), ((), ())),
            preferred_element_type=f32,
        )

    acc = dot(mov_ref[...], Wm)
    acc += tdot(onehot_t(age_ref[...], NUM_AGE), dot(aget_ref[...], Wa))
    acc += tdot(onehot_t(gdr_ref[...], NUM_GENDERS), dot(gdrt_ref[...], Wg))
    acc += tdot(onehot_t(occ_ref[...], NUM_OCC), dot(occt_ref[...], Wo))
    acc += tdot(genT_ref[...], dot(gw_ref[...], Wgen))
    acc += dot(gb_ref[...], Wgen) + b1_ref[...]
    h = jnp.maximum(acc, 0.0)
    out_ref[...] = lax.dot_general(
        w2_ref[...], h, (((1,), (1,)), ((), ())),
        preferred_element_type=f32,
    ) + b2_ref[...]


def _mlp(mov, genresT, age_r, gdr_r, occ_r,
         age_table, gender_table, occupation_table,
         genre_W, genre_b, fc1_W, fc1_b, fc2_W, fc2_b,
         interpret=False):
    row_spec = lambda d: pl.BlockSpec((d, _B_BLK), lambda i: (0, i))
    full_spec = lambda a, b: pl.BlockSpec((a, b), lambda i: (0, 0))
    out2 = pl.pallas_call(
        _mlp_body,
        grid=(_GRID,),
        in_specs=[
            pl.BlockSpec((_B_BLK, EMB), lambda i: (i, 0)),
            row_spec(NUM_GENRES),
            row_spec(1),
            row_spec(1),
            row_spec(1),
            full_spec(NUM_AGE, EMB),
            full_spec(NUM_GENDERS, EMB),
            full_spec(NUM_OCC, EMB),
            full_spec(NUM_GENRES, EMB),
            full_spec(1, EMB),
            full_spec(5 * EMB, HID),
            full_spec(1, HID),
            full_spec(1, HID),
            full_spec(1, 1),
        ],
        out_specs=pl.BlockSpec((1, _B_BLK), lambda i: (0, i)),
        out_shape=jax.ShapeDtypeStruct((1, BATCH), jnp.float32),
        interpret=interpret,
    )(mov, genresT, age_r, gdr_r, occ_r,
      age_table, gender_table, occupation_table,
      genre_W, genre_b.reshape(1, EMB),
      fc1_W, fc1_b.reshape(1, HID), fc2_W.reshape(1, HID),
      fc2_b.reshape(1, 1))
    return out2.reshape(BATCH)


def kernel(age_group, movie, gender, occupation, genres,
           movie_table, gender_table, occupation_table, age_table,
           genre_W, genre_b, fc1_W, fc1_b, fc2_W, fc2_b):
    i32 = jnp.int32
    mov = _movie_gather(movie_table, movie.astype(i32))
    return _mlp(
        mov,
        genres.astype(jnp.float32).T,
        age_group.astype(i32).reshape(1, BATCH),
        gender.astype(i32).reshape(1, BATCH),
        occupation.astype(i32).reshape(1, BATCH),
        age_table, gender_table, occupation_table,
        genre_W, genre_b, fc1_W, fc1_b, fc2_W, fc2_b,
    )
